# baseline (speedup 1.0000x reference)
import jax
import jax.numpy as jnp
from jax import lax
from jax.experimental import pallas as pl
from jax.experimental.pallas import tpu as pltpu
from jax.experimental.pallas import tpu_sc as plsc

MODEL_DIM = 768
HIDDEN_DIM = 3072
NUM_EXPERTS = 8
TOP_K = 2
SEQ = 2048

NC, NS, L = 2, 16, 16
NW = NC * NS
CHUNK = SEQ // NW
NCELL = NUM_EXPERTS * TOP_K


def _gate_kernel(x_ref, gw_ref, gb_ref, lt_ref):
    logits = jax.lax.dot_general(
        x_ref[...], gw_ref[...], (((1,), (1,)), ((), ())),
        preferred_element_type=jnp.float32) + gb_ref[...]
    lt_ref[...] = jnp.concatenate(
        [logits, jnp.full((SEQ, L - NUM_EXPERTS), -jnp.inf, jnp.float32)],
        axis=1)


def _routing_kernel(lt_hbm, p_hbm, chunk_v, vec_v, shared_v):
    wid = lax.axis_index("s") * NC + lax.axis_index("c")
    base = wid * CHUNK
    pltpu.sync_copy(lt_hbm.at[pl.ds(base, CHUNK)], chunk_v)

    iota = lax.iota(jnp.int32, L)
    ninf = jnp.full((L,), -jnp.inf, jnp.float32)

    def tok(t, bv):
        v = chunk_v[t]
        m1 = v[0]
        i1 = jnp.int32(0)
        m2 = -jnp.inf
        i2 = jnp.int32(0)
        for e in range(1, NUM_EXPERTS):
            le = v[e]
            b1 = le > m1
            b2 = le > m2
            m2n = jnp.where(b1, m1, jnp.where(b2, le, m2))
            i2n = jnp.where(b1, i1, jnp.where(b2, e, i2))
            m1 = jnp.where(b1, le, m1)
            i1 = jnp.where(b1, e, i1)
            m2, i2 = m2n, i2n
        bv = jnp.where(iota == 2 * i1, jnp.full((L,), m1, jnp.float32), bv)
        bv = jnp.where(iota == 2 * i2 + 1, jnp.full((L,), m2, jnp.float32), bv)
        return bv

    bv = jax.lax.fori_loop(0, CHUNK, tok, ninf)

    vec_v[...] = bv
    pltpu.sync_copy(vec_v, shared_v.at[wid])
    plsc.subcore_barrier()

    @pl.when(wid == 0)
    def _():
        def merge(r, acc):
            pltpu.sync_copy(shared_v.at[r], vec_v)
            rv = vec_v[...]
            return jnp.where(rv > -jnp.inf, rv, acc)

        acc = jax.lax.fori_loop(0, NW, merge, ninf)
        vec_v[...] = acc
        pltpu.sync_copy(vec_v, p_hbm)


def _expert_kernel(x8_ref, fc1_w_ref, fc1_b_ref, fc2_w_ref, fc2_b_ref, y_ref):
    def expert(e):
        h = jax.lax.dot_general(
            x8_ref[...], fc1_w_ref[e], (((1,), (1,)), ((), ())),
            preferred_element_type=jnp.float32) + fc1_b_ref[e]
        h = h * jax.nn.sigmoid(h)
        return jax.lax.dot_general(
            h, fc2_w_ref[e], (((1,), (1,)), ((), ())),
            preferred_element_type=jnp.float32) + fc2_b_ref[e]

    y_ref[0] = expert(0)
    y_ref[1] = expert(1)


def _combine_kernel(cell_ref, y_ref, out_ref):
    v0 = cell_ref[:, 0:1]
    v1 = cell_ref[:, 1:2]
    m = jnp.maximum(v0, v1)
    e0 = jnp.exp(v0 - m)
    e1 = jnp.exp(v1 - m)
    denom = e0 + e1
    out_ref[...] = jnp.full((SEQ, MODEL_DIM), jnp.nan, dtype=jnp.float32)
    out_ref[0:NUM_EXPERTS, :] = (e0 / denom) * y_ref[0] + \
        (e1 / denom) * y_ref[1]


@jax.jit
def kernel(x, fc1_w, fc1_b, fc2_w, fc2_b, gate_w, gate_b):
    B, S, D = x.shape
    x2 = x.reshape(S, D)

    logits_pad = pl.pallas_call(
        _gate_kernel,
        out_shape=jax.ShapeDtypeStruct((S, L), jnp.float32),
    )(x2, gate_w, gate_b.reshape(1, NUM_EXPERTS))

    mesh = plsc.VectorSubcoreMesh(core_axis_name="c", subcore_axis_name="s")
    cells = pl.kernel(
        _routing_kernel,
        mesh=mesh,
        out_type=jax.ShapeDtypeStruct((NCELL,), jnp.float32),
        scratch_types=[
            pltpu.VMEM((CHUNK, L), jnp.float32),
            pltpu.VMEM((NCELL,), jnp.float32),
            pltpu.VMEM_SHARED((NW, NCELL), jnp.float32),
        ],
    )(logits_pad)

    y01 = pl.pallas_call(
        _expert_kernel,
        grid=(1,),
        in_specs=[
            pl.BlockSpec((NUM_EXPERTS, D), lambda i: (0, 0)),
            pl.BlockSpec((TOP_K, HIDDEN_DIM, D), lambda i: (0, 0, 0)),
            pl.BlockSpec((TOP_K, 1, HIDDEN_DIM), lambda i: (0, 0, 0)),
            pl.BlockSpec((TOP_K, D, HIDDEN_DIM), lambda i: (0, 0, 0)),
            pl.BlockSpec((TOP_K, 1, D), lambda i: (0, 0, 0)),
        ],
        out_specs=pl.BlockSpec((TOP_K, NUM_EXPERTS, D), lambda i: (0, 0, 0)),
        out_shape=jax.ShapeDtypeStruct((TOP_K, NUM_EXPERTS, D), jnp.float32),
        compiler_params=pltpu.CompilerParams(
            vmem_limit_bytes=100 * 1024 * 1024),
    )(
        x2[0:NUM_EXPERTS, :],
        fc1_w,
        fc1_b.reshape(NUM_EXPERTS, 1, HIDDEN_DIM),
        fc2_w,
        fc2_b.reshape(NUM_EXPERTS, 1, MODEL_DIM),
    )

    out = pl.pallas_call(
        _combine_kernel,
        out_shape=jax.ShapeDtypeStruct((S, D), jnp.float32),
    )(cells.reshape(NUM_EXPERTS, TOP_K), y01)

    return out.reshape(B, S, D)


# speedup vs baseline: 1.0011x; 1.0011x over previous
import jax
import jax.numpy as jnp
from jax import lax
from jax.experimental import pallas as pl
from jax.experimental.pallas import tpu as pltpu
from jax.experimental.pallas import tpu_sc as plsc

MODEL_DIM = 768
HIDDEN_DIM = 3072
NUM_EXPERTS = 8
TOP_K = 2
SEQ = 2048

NC, NS, L = 2, 16, 16
NW = NC * NS
CHUNK = SEQ // NW
NCELL = NUM_EXPERTS * TOP_K


def _gate_kernel(x_ref, gw_ref, gb_ref, lt_ref):
    logits = jax.lax.dot_general(
        x_ref[...], gw_ref[...], (((1,), (1,)), ((), ())),
        preferred_element_type=jnp.float32) + gb_ref[...]
    lt_ref[...] = jnp.concatenate(
        [logits, jnp.full((SEQ, L - NUM_EXPERTS), -jnp.inf, jnp.float32)],
        axis=1)


def _routing_kernel(lt_hbm, p_hbm, chunk_v, vec_v, shared_v):
    wid = lax.axis_index("s") * NC + lax.axis_index("c")
    base = wid * CHUNK
    pltpu.sync_copy(lt_hbm.at[pl.ds(base, CHUNK)], chunk_v)

    iota = lax.iota(jnp.int32, L)
    ninf = jnp.full((L,), -jnp.inf, jnp.float32)

    def tok(t, bv):
        v = chunk_v[t]
        m1 = v[0]
        i1 = jnp.int32(0)
        m2 = -jnp.inf
        i2 = jnp.int32(0)
        for e in range(1, NUM_EXPERTS):
            le = v[e]
            b1 = le > m1
            b2 = le > m2
            m2n = jnp.where(b1, m1, jnp.where(b2, le, m2))
            i2n = jnp.where(b1, i1, jnp.where(b2, e, i2))
            m1 = jnp.where(b1, le, m1)
            i1 = jnp.where(b1, e, i1)
            m2, i2 = m2n, i2n
        bv = jnp.where(iota == 2 * i1, jnp.full((L,), m1, jnp.float32), bv)
        bv = jnp.where(iota == 2 * i2 + 1, jnp.full((L,), m2, jnp.float32), bv)
        return bv

    bv = jax.lax.fori_loop(0, CHUNK, tok, ninf)

    vec_v[...] = bv
    pltpu.sync_copy(vec_v, shared_v.at[wid])
    plsc.subcore_barrier()

    @pl.when(wid == 0)
    def _():
        def merge(r, acc):
            pltpu.sync_copy(shared_v.at[r], vec_v)
            rv = vec_v[...]
            return jnp.where(rv > -jnp.inf, rv, acc)

        acc = jax.lax.fori_loop(0, NW, merge, ninf)
        vec_v[...] = acc
        pltpu.sync_copy(vec_v, p_hbm)


def _expert_kernel(x8_ref, fc1_w_ref, fc1_b_ref, fc2_w_ref, fc2_b_ref, y_ref):
    def expert(e):
        h = jax.lax.dot_general(
            x8_ref[...], fc1_w_ref[e], (((1,), (1,)), ((), ())),
            preferred_element_type=jnp.float32) + fc1_b_ref[e]
        h = h * jax.nn.sigmoid(h)
        return jax.lax.dot_general(
            h, fc2_w_ref[e], (((1,), (1,)), ((), ())),
            preferred_element_type=jnp.float32) + fc2_b_ref[e]

    y_ref[0] = expert(0)
    y_ref[1] = expert(1)


def _combine_kernel(cell_ref, y_ref, out_ref):
    v0 = cell_ref[:, 0:1]
    v1 = cell_ref[:, 1:2]
    m = jnp.maximum(v0, v1)
    e0 = jnp.exp(v0 - m)
    e1 = jnp.exp(v1 - m)
    denom = e0 + e1
    out_ref[...] = jnp.full((SEQ, MODEL_DIM), jnp.nan, dtype=jnp.float32)
    out_ref[0:NUM_EXPERTS, :] = (e0 / denom) * y_ref[0] + \
        (e1 / denom) * y_ref[1]


@jax.jit
def kernel(x, fc1_w, fc1_b, fc2_w, fc2_b, gate_w, gate_b):
    B, S, D = x.shape
    x2 = x.reshape(S, D)

    logits_pad = pl.pallas_call(
        _gate_kernel,
        out_shape=jax.ShapeDtypeStruct((S, L), jnp.float32),
    )(x2, gate_w, gate_b.reshape(1, NUM_EXPERTS))

    y01 = pl.pallas_call(
        _expert_kernel,
        grid=(1,),
        in_specs=[
            pl.BlockSpec((NUM_EXPERTS, D), lambda i: (0, 0)),
            pl.BlockSpec((TOP_K, HIDDEN_DIM, D), lambda i: (0, 0, 0)),
            pl.BlockSpec((TOP_K, 1, HIDDEN_DIM), lambda i: (0, 0, 0)),
            pl.BlockSpec((TOP_K, D, HIDDEN_DIM), lambda i: (0, 0, 0)),
            pl.BlockSpec((TOP_K, 1, D), lambda i: (0, 0, 0)),
        ],
        out_specs=pl.BlockSpec((TOP_K, NUM_EXPERTS, D), lambda i: (0, 0, 0)),
        out_shape=jax.ShapeDtypeStruct((TOP_K, NUM_EXPERTS, D), jnp.float32),
        compiler_params=pltpu.CompilerParams(
            vmem_limit_bytes=100 * 1024 * 1024),
    )(
        x2[0:NUM_EXPERTS, :],
        fc1_w,
        fc1_b.reshape(NUM_EXPERTS, 1, HIDDEN_DIM),
        fc2_w,
        fc2_b.reshape(NUM_EXPERTS, 1, MODEL_DIM),
    )

    mesh = plsc.VectorSubcoreMesh(core_axis_name="c", subcore_axis_name="s")
    cells = pl.kernel(
        _routing_kernel,
        mesh=mesh,
        out_type=jax.ShapeDtypeStruct((NCELL,), jnp.float32),
        scratch_types=[
            pltpu.VMEM((CHUNK, L), jnp.float32),
            pltpu.VMEM((NCELL,), jnp.float32),
            pltpu.VMEM_SHARED((NW, NCELL), jnp.float32),
        ],
    )(logits_pad)

    out = pl.pallas_call(
        _combine_kernel,
        out_shape=jax.ShapeDtypeStruct((S, D), jnp.float32),
    )(cells.reshape(NUM_EXPERTS, TOP_K), y01)

    return out.reshape(B, S, D)
